# Initial kernel scaffold; baseline (speedup 1.0000x reference)
import jax
import jax.numpy as jnp
from jax.experimental import pallas as pl


def kernel(x, table, W1, b1, W2, b2, W3, b3):
    raise NotImplementedError("write your pallas kernel here")


# speedup vs baseline: 7.2524x; 7.2524x over previous
import functools

import numpy as np
import jax
import jax.numpy as jnp
from jax import lax
from jax.experimental import pallas as pl
from jax.experimental.pallas import tpu as pltpu
from jax.experimental.pallas import tpu_sc as plsc

_B = 16384
_F = 26
_D = 16
_OFF = np.concatenate([[0], np.cumsum([100000] * _F)[:-1]]).astype(np.int32)

_NC = 2
_NS = 16
_NW = _NC * _NS
_ROWS_PER_W = _B * _F // _NW
_CHUNK = 1664
_NCHUNK = _ROWS_PER_W // _CHUNK


def _gather_body(idx_hbm, table_hbm, out_hbm, idx_v, rows_v, sem):
    wid = lax.axis_index("s") * _NC + lax.axis_index("c")
    base = wid * _ROWS_PER_W

    def chunk(c, carry):
        off = base + c * _CHUNK
        pltpu.sync_copy(idx_hbm.at[pl.ds(off, _CHUNK)], idx_v)
        pltpu.async_copy(table_hbm.at[idx_v], rows_v, sem).wait()
        pltpu.sync_copy(rows_v, out_hbm.at[pl.ds(off, _CHUNK)])
        return carry

    lax.fori_loop(0, _NCHUNK, chunk, 0)


_gather = functools.partial(
    pl.kernel,
    out_type=jax.ShapeDtypeStruct((_B * _F, _D), jnp.float32),
    mesh=plsc.VectorSubcoreMesh(core_axis_name="c", subcore_axis_name="s"),
    scratch_types=[
        pltpu.VMEM((_CHUNK,), jnp.int32),
        pltpu.VMEM((_CHUNK, _D), jnp.float32),
        pltpu.SemaphoreType.DMA,
    ],
    compiler_params=pltpu.CompilerParams(use_tc_tiling_on_sc=False),
)(_gather_body)


_BLK = 512


def _mlp_body(e_ref, w1_ref, b1_ref, w2_ref, b2_ref, w3_ref, b3_ref, o_ref):
    h = e_ref[...]
    h1 = jnp.maximum(
        jnp.dot(h, w1_ref[...], preferred_element_type=jnp.float32) + b1_ref[...], 0.0)
    h2 = jnp.maximum(
        jnp.dot(h1, w2_ref[...], preferred_element_type=jnp.float32) + b2_ref[...], 0.0)
    logits = jnp.dot(h2, w3_ref[...], preferred_element_type=jnp.float32) + b3_ref[...]
    o_ref[...] = 1.0 / (1.0 + jnp.exp(-logits))


def _mlp(e2, W1, b1, W2, b2, W3, b3):
    grid = (_B // _BLK,)
    return pl.pallas_call(
        _mlp_body,
        grid=grid,
        in_specs=[
            pl.BlockSpec((_BLK, _F * _D), lambda i: (i, 0)),
            pl.BlockSpec((_F * _D, 4), lambda i: (0, 0)),
            pl.BlockSpec((1, 4), lambda i: (0, 0)),
            pl.BlockSpec((4, 4), lambda i: (0, 0)),
            pl.BlockSpec((1, 4), lambda i: (0, 0)),
            pl.BlockSpec((4, 1), lambda i: (0, 0)),
            pl.BlockSpec((1, 1), lambda i: (0, 0)),
        ],
        out_specs=pl.BlockSpec((_BLK, 1), lambda i: (i, 0)),
        out_shape=jax.ShapeDtypeStruct((_B, 1), jnp.float32),
    )(e2, W1, b1, W2, b2, W3, b3)


def kernel(x, table, W1, b1, W2, b2, W3, b3):
    idx = (x + jnp.asarray(_OFF)[None, :]).reshape(-1)
    embed = _gather(idx, table)
    e2 = embed.reshape(_B, _F * _D)
    out = _mlp(e2, W1, b1.reshape(1, 4), W2, b2.reshape(1, 4),
               W3, b3.reshape(1, 1))
    return out.reshape(_B)


# speedup vs baseline: 12.1212x; 1.6713x over previous
import functools

import numpy as np
import jax
import jax.numpy as jnp
from jax import lax
from jax.experimental import pallas as pl
from jax.experimental.pallas import tpu as pltpu
from jax.experimental.pallas import tpu_sc as plsc

_B = 16384
_F = 26
_D = 16
_RT = 2600000
_FS = 100000
_OFF = np.concatenate([[0], np.cumsum([_FS] * _F)[:-1]]).astype(np.int32)

_CB = 8192
_NBLK = (_RT + _CB - 1) // _CB
_RTP = _NBLK * _CB


def _tprime_body(tt_ref, w1e_ref, o_ref):
    i = pl.program_id(0)
    r0 = i * _CB
    f0 = r0 // _FS
    bnd = (f0 + 1) * _FS
    I = tt_ref[...]
    W0 = w1e_ref[pl.ds(f0 * _D, _D), :]
    W1b = w1e_ref[pl.ds(jnp.minimum(f0 + 1, _F) * _D, _D), :]
    TA = lax.dot_general(I, W0, (((0,), (0,)), ((), ())),
                         preferred_element_type=jnp.float32)
    TB = lax.dot_general(I, W1b, (((0,), (0,)), ((), ())),
                         preferred_element_type=jnp.float32)
    rid = r0 + lax.broadcasted_iota(jnp.int32, (_CB, 1), 0)
    T = jnp.where(rid < bnd, TA, TB)
    for j in range(8):
        o_ref[:, 16 * j:16 * (j + 1)] = T[1024 * j:1024 * (j + 1), :]


def _tprime(tt, w1e):
    return pl.pallas_call(
        _tprime_body,
        grid=(_NBLK,),
        in_specs=[
            pl.BlockSpec((_D, _CB), lambda i: (0, i)),
            pl.BlockSpec(((_F + 1) * _D, _D), lambda i: (0, 0)),
        ],
        out_specs=pl.BlockSpec((_CB // 8, 128), lambda i: (i, 0)),
        out_shape=jax.ShapeDtypeStruct((_RTP // 8, 128), jnp.float32),
    )(tt, w1e)


_NC = 2
_NS = 16
_NW = _NC * _NS
_BPW = _B // _NW
_CHB = 128
_NCH = _BPW // _CHB
_ROWS = _CHB * _F

_i16 = lambda: lax.iota(jnp.int32, 16)


def _take(v, idx):
    return jnp.take(v, idx)


def _scmlp_body(idx_hbm, tp_hbm, wv_hbm, out_hbm,
                idx_v, rows_v, wv_v, out_v, sem):
    wid = lax.axis_index("s") * _NC + lax.axis_index("c")
    base_rows = wid * _BPW * _F
    base_out = wid * _BPW
    pltpu.sync_copy(wv_hbm, wv_v)
    b1v = wv_v[0, :]
    w2f = wv_v[1, :]
    b2v = wv_v[2, :]
    w3v = wv_v[3, :]
    b3v = wv_v[4, :]
    lanes = _i16()
    perm_rep = lanes & 3
    perm1 = (lanes + 1) & 15
    perm2 = (lanes + 2) & 15
    perm4 = (lanes + 4) & 15
    perm8 = (lanes + 8) & 15

    for c in range(_NCH):
        off = base_rows + c * _ROWS
        pltpu.sync_copy(idx_hbm.at[pl.ds(off, _ROWS)], idx_v)
        pltpu.async_copy(tp_hbm.at[idx_v], rows_v, sem).wait()

        def batch_body(bb, acc):
            s = rows_v[bb * _F + 0, :]
            for t in range(1, _F):
                s = s + rows_v[bb * _F + t, :]
            h1 = jnp.maximum(s + b1v, 0.0)
            hrep = _take(h1, perm_rep)
            p = hrep * w2f
            g = p + _take(p, perm1)
            g = g + _take(g, perm2)
            h2 = jnp.maximum(g + b2v, 0.0)
            p2 = h2 * w3v
            t_ = p2 + _take(p2, perm8)
            t_ = t_ + _take(t_, perm4)
            t_ = t_ + _take(t_, perm2)
            t_ = t_ + _take(t_, perm1)
            logit = t_ + b3v
            sig = 1.0 / (1.0 + jnp.exp(-logit))
            return jnp.where(lanes == (bb & 15), sig, acc)

        def group_body(g, carry):
            acc = jnp.zeros((16,), jnp.float32)
            acc = lax.fori_loop(g * 16, (g + 1) * 16,
                                lambda bb, a: batch_body(bb, a), acc)
            out_v[pl.ds(c * _CHB + g * 16, 16)] = acc
            return carry

        lax.fori_loop(0, _CHB // 16, group_body, 0)

    pltpu.sync_copy(out_v, out_hbm.at[pl.ds(base_out, _BPW)])


_scmlp = functools.partial(
    pl.kernel,
    out_type=jax.ShapeDtypeStruct((_B,), jnp.float32),
    mesh=plsc.VectorSubcoreMesh(core_axis_name="c", subcore_axis_name="s"),
    scratch_types=[
        pltpu.VMEM((_ROWS,), jnp.int32),
        pltpu.VMEM((_ROWS, _D), jnp.float32),
        pltpu.VMEM((5, 16), jnp.float32),
        pltpu.VMEM((_BPW,), jnp.float32),
        pltpu.SemaphoreType.DMA,
    ],
    compiler_params=pltpu.CompilerParams(use_tc_tiling_on_sc=False),
)(_scmlp_body)


def kernel(x, table, W1, b1, W2, b2, W3, b3):
    r = (x + jnp.asarray(_OFF)[None, :]).reshape(-1)
    rem = r % _CB
    idx = 8 * (1024 * (r // _CB) + rem % 1024) + rem // 1024
    w1e = jnp.zeros(((_F + 1) * _D, _D), jnp.float32)
    w1e = w1e.at[: _F * _D, :4].set(W1)
    tp = _tprime(table.T, w1e)
    tp = tp.reshape(_RTP, _D)
    wv = jnp.zeros((5, 16), jnp.float32)
    wv = wv.at[0, :4].set(b1)
    wv = wv.at[1, :].set(W2.T.reshape(16))
    wv = wv.at[2, ::4].set(b2)
    wv = wv.at[3, ::4].set(W3[:, 0])
    wv = wv.at[4, :].set(b3[0])
    return _scmlp(idx, tp, wv)
